# baseline (speedup 1.0000x reference)
import functools

import jax
import jax.numpy as jnp
from jax import lax
from jax.experimental import pallas as pl
from jax.experimental.pallas import tpu as pltpu
from jax.experimental.pallas import tpu_sc as plsc

B = 32
N = 1048576
NBINS = 100
OUTPAD = 128
CHUNK = 16384
N_SC = 491520
N_TC = N - N_SC
NCHUNKS = N_SC // CHUNK
VPC = CHUNK // 16
NC = 2
GRP = 7
BIN_POS = [0, 12, 25, 37, 50, 62, 75, 87, 99]
_LE_THRESH = (-4.0, -3.0, -2.0, -1.0)
_LT_THRESH = (1.0, 2.0, 3.0, 4.0)
CB = 8192
G_TC = N_TC // CB
TC0 = N_SC // CB


def _hist_body(x_hbm, out_hbm, buf0, buf1, row_v, sem0, sem1):
  wid = lax.axis_index("s") * NC + lax.axis_index("c")
  iota = lax.iota(jnp.int32, 16)
  one = jnp.ones((16,), jnp.int32)
  zi = jnp.zeros((16,), jnp.int32)
  m63 = jnp.full((16,), 63, jnp.int32)

  def pack_one(acc, v):
    c = jnp.minimum(jnp.maximum(v.astype(jnp.int32), -4), 4)
    return acc + (one << (c * 3 + 12))

  def fold_l2(l2, acc):
    l2e, l2o = l2
    return (l2e + (acc & MASK_E), l2o + ((acc >> 3) & MASK_O))

  def unpack_l2(wides, l2):
    l2e, l2o = l2
    new = list(wides)
    for k2 in range(5):
      new[2 * k2] = new[2 * k2] + ((l2e >> (6 * k2)) & m63)
    for k2 in range(4):
      new[2 * k2 + 1] = new[2 * k2 + 1] + ((l2o >> (6 * k2)) & m63)
    return tuple(new)

  def start_copy(buf, sem, ci):
    off = jnp.minimum(ci, NCHUNKS - 1) * CHUNK
    pltpu.make_async_copy(
        x_hbm.at[wid, pl.ds(off, CHUNK)], buf, sem).start()

  def wait_copy(buf, sem):
    pltpu.make_async_copy(
        x_hbm.at[wid, pl.ds(0, CHUNK)], buf, sem).wait()

  seven = jnp.full((16,), 7, jnp.int32)

  def unpack_into(wides, acc):
    return tuple(w + ((acc >> (3 * k)) & seven)
                 for k, w in enumerate(wides))

  NG = VPC // GRP
  REM2 = VPC - NG * GRP

  def process_chunk(buf, wides):
    @plsc.parallel_loop(0, NG, carry=wides, unroll=2)
    def chunk_loop(g, wides):
      base = g * (GRP * 16)
      acc_a = zi
      acc_b = zi
      for u in range(GRP):
        if u % 2 == 0:
          acc_a = pack_one(acc_a, buf[pl.ds(base + u * 16, 16)])
        else:
          acc_b = pack_one(acc_b, buf[pl.ds(base + u * 16, 16)])
      return unpack_into(wides, acc_a + acc_b)

    wides = chunk_loop
    acc = zi
    for u in range(REM2):
      acc = pack_one(acc, buf[pl.ds((NG * GRP + u) * 16, 16)])
    return unpack_into(wides, acc)

  start_copy(buf0, sem0, 0)
  start_copy(buf1, sem1, 1)

  wides = tuple(zi for _ in range(9))

  @pl.loop(0, NCHUNKS // 2, init_carry=wides)
  def pair_loop(p, wides):
    ci = p * 2
    wait_copy(buf0, sem0)
    wides = process_chunk(buf0, wides)
    start_copy(buf0, sem0, ci + 2)
    wait_copy(buf1, sem1)
    wides = process_chunk(buf1, wides)
    start_copy(buf1, sem1, ci + 3)
    return wides

  wides = pair_loop
  wait_copy(buf0, sem0)
  wait_copy(buf1, sem1)

  def lane_sum(a):
    for sh in (1, 2, 4, 8):
      perm = iota ^ sh
      a = a + jnp.take_along_axis(a, perm, axis=0, mode="promise_in_bounds")
    return a

  df = [lane_sum(w).astype(jnp.float32) for w in wides]

  zf = jnp.zeros((16,), jnp.float32)
  for j in range(OUTPAD // 16):
    vreg = zf
    for k, p in enumerate(BIN_POS):
      if j * 16 <= p < (j + 1) * 16:
        vreg = jnp.where(iota == (p - j * 16), df[k], vreg)
    row_v[pl.ds(j * 16, 16)] = vreg

  pltpu.sync_copy(row_v, out_hbm.at[wid])


def _tc_body(x_ref, out_ref, *wides):
  i = pl.program_id(0)

  @pl.when(i == 0)
  def _():
    for w in wides:
      w[...] = jnp.zeros((B, 128), jnp.int32)

  xb = x_ref[...].astype(jnp.int32)
  c = jnp.minimum(jnp.maximum(xb, -4), 4)
  sh = c * 3 + 12
  ones = jnp.ones((B, CB), jnp.int32)
  packed = ones << sh
  NSL = CB // 128
  for g in range((NSL + GRP - 1) // GRP):
    lo = g * GRP
    hi = min(lo + GRP, NSL)
    acc = packed[:, lo * 128:(lo + 1) * 128]
    for j in range(lo + 1, hi):
      acc = acc + packed[:, j * 128:(j + 1) * 128]
    for k, w in enumerate(wides):
      w[...] += (acc >> (3 * k)) & 7

  @pl.when(i == G_TC - 1)
  def _():
    s = [jnp.sum(w[...], axis=1, keepdims=True).astype(jnp.float32)
         for w in wides]
    iot = lax.broadcasted_iota(jnp.int32, (B, 128), 1)
    o = jnp.zeros((B, 128), jnp.float32)
    for k, p in enumerate(BIN_POS):
      o = jnp.where(iot == p, s[k], o)
    out_ref[...] = o


def _tc_call(x):
  return pl.pallas_call(
      _tc_body,
      grid=(G_TC,),
      in_specs=[pl.BlockSpec((B, CB), lambda i: (0, TC0 + i))],
      out_specs=pl.BlockSpec((B, 128), lambda i: (0, 0)),
      out_shape=jax.ShapeDtypeStruct((B, 128), jnp.float32),
      scratch_shapes=[pltpu.VMEM((B, 128), jnp.int32)] * 9,
      compiler_params=pltpu.CompilerParams(
          dimension_semantics=("arbitrary",)),
  )(x)


@jax.jit
def kernel(x):
  mesh = plsc.VectorSubcoreMesh(core_axis_name="c", subcore_axis_name="s")
  sc_out = pl.kernel(
      _hist_body,
      out_type=jax.ShapeDtypeStruct((B, OUTPAD), jnp.float32),
      mesh=mesh,
      scratch_types=[
          pltpu.VMEM((CHUNK,), jnp.float32),
          pltpu.VMEM((CHUNK,), jnp.float32),
          pltpu.VMEM((OUTPAD,), jnp.float32),
          pltpu.SemaphoreType.DMA,
          pltpu.SemaphoreType.DMA,
      ],
  )(x)
  tc_out = _tc_call(x)
  return (sc_out + tc_out)[:, :NBINS]


# speedup vs baseline: 1.0530x; 1.0530x over previous
import functools

import jax
import jax.numpy as jnp
from jax import lax
from jax.experimental import pallas as pl
from jax.experimental.pallas import tpu as pltpu
from jax.experimental.pallas import tpu_sc as plsc

B = 32
N = 1048576
NBINS = 100
OUTPAD = 128
CHUNK = 16384
N_SC = 458752
N_TC = N - N_SC
NCHUNKS = N_SC // CHUNK
VPC = CHUNK // 16
NC = 2
GRP = 7
BIN_POS = [0, 12, 25, 37, 50, 62, 75, 87, 99]
_LE_THRESH = (-4.0, -3.0, -2.0, -1.0)
_LT_THRESH = (1.0, 2.0, 3.0, 4.0)
CB = 8192
G_TC = N_TC // CB
TC0 = N_SC // CB


def _hist_body(x_hbm, out_hbm, buf0, buf1, row_v, sem0, sem1):
  wid = lax.axis_index("s") * NC + lax.axis_index("c")
  iota = lax.iota(jnp.int32, 16)
  one = jnp.ones((16,), jnp.int32)
  zi = jnp.zeros((16,), jnp.int32)
  m63 = jnp.full((16,), 63, jnp.int32)

  def pack_one(acc, v):
    c = jnp.minimum(jnp.maximum(v.astype(jnp.int32), -4), 4)
    return acc + (one << (c * 3 + 12))

  def fold_l2(l2, acc):
    l2e, l2o = l2
    return (l2e + (acc & MASK_E), l2o + ((acc >> 3) & MASK_O))

  def unpack_l2(wides, l2):
    l2e, l2o = l2
    new = list(wides)
    for k2 in range(5):
      new[2 * k2] = new[2 * k2] + ((l2e >> (6 * k2)) & m63)
    for k2 in range(4):
      new[2 * k2 + 1] = new[2 * k2 + 1] + ((l2o >> (6 * k2)) & m63)
    return tuple(new)

  def start_copy(buf, sem, ci):
    off = jnp.minimum(ci, NCHUNKS - 1) * CHUNK
    pltpu.make_async_copy(
        x_hbm.at[wid, pl.ds(off, CHUNK)], buf, sem).start()

  def wait_copy(buf, sem):
    pltpu.make_async_copy(
        x_hbm.at[wid, pl.ds(0, CHUNK)], buf, sem).wait()

  seven = jnp.full((16,), 7, jnp.int32)

  def unpack_into(wides, acc):
    return tuple(w + ((acc >> (3 * k)) & seven)
                 for k, w in enumerate(wides))

  NG = VPC // GRP
  REM2 = VPC - NG * GRP

  def process_chunk(buf, wides):
    @plsc.parallel_loop(0, NG, carry=wides, unroll=2)
    def chunk_loop(g, wides):
      base = g * (GRP * 16)
      acc_a = zi
      acc_b = zi
      for u in range(GRP):
        if u % 2 == 0:
          acc_a = pack_one(acc_a, buf[pl.ds(base + u * 16, 16)])
        else:
          acc_b = pack_one(acc_b, buf[pl.ds(base + u * 16, 16)])
      return unpack_into(wides, acc_a + acc_b)

    wides = chunk_loop
    acc = zi
    for u in range(REM2):
      acc = pack_one(acc, buf[pl.ds((NG * GRP + u) * 16, 16)])
    return unpack_into(wides, acc)

  start_copy(buf0, sem0, 0)
  start_copy(buf1, sem1, 1)

  wides = tuple(zi for _ in range(9))

  @pl.loop(0, NCHUNKS // 2, init_carry=wides)
  def pair_loop(p, wides):
    ci = p * 2
    wait_copy(buf0, sem0)
    wides = process_chunk(buf0, wides)
    start_copy(buf0, sem0, ci + 2)
    wait_copy(buf1, sem1)
    wides = process_chunk(buf1, wides)
    start_copy(buf1, sem1, ci + 3)
    return wides

  wides = pair_loop
  wait_copy(buf0, sem0)
  wait_copy(buf1, sem1)

  def lane_sum(a):
    for sh in (1, 2, 4, 8):
      perm = iota ^ sh
      a = a + jnp.take_along_axis(a, perm, axis=0, mode="promise_in_bounds")
    return a

  df = [lane_sum(w).astype(jnp.float32) for w in wides]

  zf = jnp.zeros((16,), jnp.float32)
  for j in range(OUTPAD // 16):
    vreg = zf
    for k, p in enumerate(BIN_POS):
      if j * 16 <= p < (j + 1) * 16:
        vreg = jnp.where(iota == (p - j * 16), df[k], vreg)
    row_v[pl.ds(j * 16, 16)] = vreg

  pltpu.sync_copy(row_v, out_hbm.at[wid])


def _tc_body(x_ref, out_ref, *wides):
  i = pl.program_id(0)

  @pl.when(i == 0)
  def _():
    for w in wides:
      w[...] = jnp.zeros((B, 128), jnp.int32)

  ones = jnp.ones((B, 128), jnp.int32)

  def pack_slice(j):
    sl = x_ref[:, j * 128:(j + 1) * 128].astype(jnp.int32)
    c = jnp.minimum(jnp.maximum(sl, -4), 4)
    return ones << (c * 3 + 12)

  NSL = CB // 128
  for g in range((NSL + GRP - 1) // GRP):
    lo = g * GRP
    hi = min(lo + GRP, NSL)
    acc = pack_slice(lo)
    for j in range(lo + 1, hi):
      acc = acc + pack_slice(j)
    for k, w in enumerate(wides):
      w[...] += (acc >> (3 * k)) & 7

  @pl.when(i == G_TC - 1)
  def _():
    s = [jnp.sum(w[...], axis=1, keepdims=True).astype(jnp.float32)
         for w in wides]
    iot = lax.broadcasted_iota(jnp.int32, (B, 128), 1)
    o = jnp.zeros((B, 128), jnp.float32)
    for k, p in enumerate(BIN_POS):
      o = jnp.where(iot == p, s[k], o)
    out_ref[...] = o


def _tc_call(x):
  return pl.pallas_call(
      _tc_body,
      grid=(G_TC,),
      in_specs=[pl.BlockSpec((B, CB), lambda i: (0, TC0 + i))],
      out_specs=pl.BlockSpec((B, 128), lambda i: (0, 0)),
      out_shape=jax.ShapeDtypeStruct((B, 128), jnp.float32),
      scratch_shapes=[pltpu.VMEM((B, 128), jnp.int32)] * 9,
      compiler_params=pltpu.CompilerParams(
          dimension_semantics=("arbitrary",)),
  )(x)


@jax.jit
def kernel(x):
  mesh = plsc.VectorSubcoreMesh(core_axis_name="c", subcore_axis_name="s")
  sc_out = pl.kernel(
      _hist_body,
      out_type=jax.ShapeDtypeStruct((B, OUTPAD), jnp.float32),
      mesh=mesh,
      scratch_types=[
          pltpu.VMEM((CHUNK,), jnp.float32),
          pltpu.VMEM((CHUNK,), jnp.float32),
          pltpu.VMEM((OUTPAD,), jnp.float32),
          pltpu.SemaphoreType.DMA,
          pltpu.SemaphoreType.DMA,
      ],
  )(x)
  tc_out = _tc_call(x)
  return (sc_out + tc_out)[:, :NBINS]


# speedup vs baseline: 1.1122x; 1.0562x over previous
import functools

import jax
import jax.numpy as jnp
from jax import lax
from jax.experimental import pallas as pl
from jax.experimental.pallas import tpu as pltpu
from jax.experimental.pallas import tpu_sc as plsc

B = 32
N = 1048576
NBINS = 100
OUTPAD = 128
CHUNK = 16384
N_SC = 425984
N_TC = N - N_SC
NCHUNKS = N_SC // CHUNK
VPC = CHUNK // 16
NC = 2
GRP = 7
BIN_POS = [0, 12, 25, 37, 50, 62, 75, 87, 99]
_LE_THRESH = (-4.0, -3.0, -2.0, -1.0)
_LT_THRESH = (1.0, 2.0, 3.0, 4.0)
CB = 8192
G_TC = N_TC // CB
TC0 = N_SC // CB


def _hist_body(x_hbm, out_hbm, buf0, buf1, row_v, sem0, sem1):
  wid = lax.axis_index("s") * NC + lax.axis_index("c")
  iota = lax.iota(jnp.int32, 16)
  one = jnp.ones((16,), jnp.int32)
  zi = jnp.zeros((16,), jnp.int32)
  m63 = jnp.full((16,), 63, jnp.int32)

  def pack_one(acc, v):
    c = jnp.minimum(jnp.maximum(v.astype(jnp.int32), -4), 4)
    return acc + (one << (c * 3 + 12))

  def fold_l2(l2, acc):
    l2e, l2o = l2
    return (l2e + (acc & MASK_E), l2o + ((acc >> 3) & MASK_O))

  def unpack_l2(wides, l2):
    l2e, l2o = l2
    new = list(wides)
    for k2 in range(5):
      new[2 * k2] = new[2 * k2] + ((l2e >> (6 * k2)) & m63)
    for k2 in range(4):
      new[2 * k2 + 1] = new[2 * k2 + 1] + ((l2o >> (6 * k2)) & m63)
    return tuple(new)

  def start_copy(buf, sem, ci):
    off = jnp.minimum(ci, NCHUNKS - 1) * CHUNK
    pltpu.make_async_copy(
        x_hbm.at[wid, pl.ds(off, CHUNK)], buf, sem).start()

  def wait_copy(buf, sem):
    pltpu.make_async_copy(
        x_hbm.at[wid, pl.ds(0, CHUNK)], buf, sem).wait()

  seven = jnp.full((16,), 7, jnp.int32)

  def unpack_into(wides, acc):
    return tuple(w + ((acc >> (3 * k)) & seven)
                 for k, w in enumerate(wides))

  NG = VPC // GRP
  REM2 = VPC - NG * GRP

  def process_chunk(buf, wides):
    @plsc.parallel_loop(0, NG, carry=wides, unroll=2)
    def chunk_loop(g, wides):
      base = g * (GRP * 16)
      acc_a = zi
      acc_b = zi
      for u in range(GRP):
        if u % 2 == 0:
          acc_a = pack_one(acc_a, buf[pl.ds(base + u * 16, 16)])
        else:
          acc_b = pack_one(acc_b, buf[pl.ds(base + u * 16, 16)])
      return unpack_into(wides, acc_a + acc_b)

    wides = chunk_loop
    acc = zi
    for u in range(REM2):
      acc = pack_one(acc, buf[pl.ds((NG * GRP + u) * 16, 16)])
    return unpack_into(wides, acc)

  start_copy(buf0, sem0, 0)
  start_copy(buf1, sem1, 1)

  wides = tuple(zi for _ in range(9))

  @pl.loop(0, NCHUNKS // 2, init_carry=wides)
  def pair_loop(p, wides):
    ci = p * 2
    wait_copy(buf0, sem0)
    wides = process_chunk(buf0, wides)
    start_copy(buf0, sem0, ci + 2)
    wait_copy(buf1, sem1)
    wides = process_chunk(buf1, wides)
    start_copy(buf1, sem1, ci + 3)
    return wides

  wides = pair_loop
  wait_copy(buf0, sem0)
  wait_copy(buf1, sem1)

  def lane_sum(a):
    for sh in (1, 2, 4, 8):
      perm = iota ^ sh
      a = a + jnp.take_along_axis(a, perm, axis=0, mode="promise_in_bounds")
    return a

  df = [lane_sum(w).astype(jnp.float32) for w in wides]

  zf = jnp.zeros((16,), jnp.float32)
  for j in range(OUTPAD // 16):
    vreg = zf
    for k, p in enumerate(BIN_POS):
      if j * 16 <= p < (j + 1) * 16:
        vreg = jnp.where(iota == (p - j * 16), df[k], vreg)
    row_v[pl.ds(j * 16, 16)] = vreg

  pltpu.sync_copy(row_v, out_hbm.at[wid])


def _tc_body(x_ref, out_ref, *wides):
  i = pl.program_id(0)

  @pl.when(i == 0)
  def _():
    for w in wides:
      w[...] = jnp.zeros((B, 128), jnp.int32)

  ones = jnp.ones((B, 128), jnp.int32)

  def pack_slice(j):
    sl = x_ref[:, j * 128:(j + 1) * 128].astype(jnp.int32)
    c = jnp.minimum(jnp.maximum(sl, -4), 4)
    return ones << (c * 3 + 12)

  NSL = CB // 128
  for g in range((NSL + GRP - 1) // GRP):
    lo = g * GRP
    hi = min(lo + GRP, NSL)
    acc = pack_slice(lo)
    for j in range(lo + 1, hi):
      acc = acc + pack_slice(j)
    for k, w in enumerate(wides):
      w[...] += (acc >> (3 * k)) & 7

  @pl.when(i == G_TC - 1)
  def _():
    s = [jnp.sum(w[...], axis=1, keepdims=True).astype(jnp.float32)
         for w in wides]
    iot = lax.broadcasted_iota(jnp.int32, (B, 128), 1)
    o = jnp.zeros((B, 128), jnp.float32)
    for k, p in enumerate(BIN_POS):
      o = jnp.where(iot == p, s[k], o)
    out_ref[...] = o


def _tc_call(x):
  return pl.pallas_call(
      _tc_body,
      grid=(G_TC,),
      in_specs=[pl.BlockSpec((B, CB), lambda i: (0, TC0 + i))],
      out_specs=pl.BlockSpec((B, 128), lambda i: (0, 0)),
      out_shape=jax.ShapeDtypeStruct((B, 128), jnp.float32),
      scratch_shapes=[pltpu.VMEM((B, 128), jnp.int32)] * 9,
      compiler_params=pltpu.CompilerParams(
          dimension_semantics=("arbitrary",)),
  )(x)


@jax.jit
def kernel(x):
  mesh = plsc.VectorSubcoreMesh(core_axis_name="c", subcore_axis_name="s")
  sc_out = pl.kernel(
      _hist_body,
      out_type=jax.ShapeDtypeStruct((B, OUTPAD), jnp.float32),
      mesh=mesh,
      scratch_types=[
          pltpu.VMEM((CHUNK,), jnp.float32),
          pltpu.VMEM((CHUNK,), jnp.float32),
          pltpu.VMEM((OUTPAD,), jnp.float32),
          pltpu.SemaphoreType.DMA,
          pltpu.SemaphoreType.DMA,
      ],
  )(x)
  tc_out = _tc_call(x)
  return (sc_out + tc_out)[:, :NBINS]


# speedup vs baseline: 1.1436x; 1.0282x over previous
import functools

import jax
import jax.numpy as jnp
from jax import lax
from jax.experimental import pallas as pl
from jax.experimental.pallas import tpu as pltpu
from jax.experimental.pallas import tpu_sc as plsc

B = 32
N = 1048576
NBINS = 100
OUTPAD = 128
CHUNK = 16384
N_SC = 393216
N_TC = N - N_SC
NCHUNKS = N_SC // CHUNK
VPC = CHUNK // 16
NC = 2
GRP = 7
BIN_POS = [0, 12, 25, 37, 50, 62, 75, 87, 99]
_LE_THRESH = (-4.0, -3.0, -2.0, -1.0)
_LT_THRESH = (1.0, 2.0, 3.0, 4.0)
CB = 8192
G_TC = N_TC // CB
TC0 = N_SC // CB


def _hist_body(x_hbm, out_hbm, buf0, buf1, row_v, sem0, sem1):
  wid = lax.axis_index("s") * NC + lax.axis_index("c")
  iota = lax.iota(jnp.int32, 16)
  one = jnp.ones((16,), jnp.int32)
  zi = jnp.zeros((16,), jnp.int32)
  m63 = jnp.full((16,), 63, jnp.int32)

  def pack_one(acc, v):
    c = jnp.minimum(jnp.maximum(v.astype(jnp.int32), -4), 4)
    return acc + (one << (c * 3 + 12))

  def fold_l2(l2, acc):
    l2e, l2o = l2
    return (l2e + (acc & MASK_E), l2o + ((acc >> 3) & MASK_O))

  def unpack_l2(wides, l2):
    l2e, l2o = l2
    new = list(wides)
    for k2 in range(5):
      new[2 * k2] = new[2 * k2] + ((l2e >> (6 * k2)) & m63)
    for k2 in range(4):
      new[2 * k2 + 1] = new[2 * k2 + 1] + ((l2o >> (6 * k2)) & m63)
    return tuple(new)

  def start_copy(buf, sem, ci):
    off = jnp.minimum(ci, NCHUNKS - 1) * CHUNK
    pltpu.make_async_copy(
        x_hbm.at[wid, pl.ds(off, CHUNK)], buf, sem).start()

  def wait_copy(buf, sem):
    pltpu.make_async_copy(
        x_hbm.at[wid, pl.ds(0, CHUNK)], buf, sem).wait()

  seven = jnp.full((16,), 7, jnp.int32)

  def unpack_into(wides, acc):
    return tuple(w + ((acc >> (3 * k)) & seven)
                 for k, w in enumerate(wides))

  NG = VPC // GRP
  REM2 = VPC - NG * GRP

  def process_chunk(buf, wides):
    @plsc.parallel_loop(0, NG, carry=wides, unroll=2)
    def chunk_loop(g, wides):
      base = g * (GRP * 16)
      acc_a = zi
      acc_b = zi
      for u in range(GRP):
        if u % 2 == 0:
          acc_a = pack_one(acc_a, buf[pl.ds(base + u * 16, 16)])
        else:
          acc_b = pack_one(acc_b, buf[pl.ds(base + u * 16, 16)])
      return unpack_into(wides, acc_a + acc_b)

    wides = chunk_loop
    acc = zi
    for u in range(REM2):
      acc = pack_one(acc, buf[pl.ds((NG * GRP + u) * 16, 16)])
    return unpack_into(wides, acc)

  start_copy(buf0, sem0, 0)
  start_copy(buf1, sem1, 1)

  wides = tuple(zi for _ in range(9))

  @pl.loop(0, NCHUNKS // 2, init_carry=wides)
  def pair_loop(p, wides):
    ci = p * 2
    wait_copy(buf0, sem0)
    wides = process_chunk(buf0, wides)
    start_copy(buf0, sem0, ci + 2)
    wait_copy(buf1, sem1)
    wides = process_chunk(buf1, wides)
    start_copy(buf1, sem1, ci + 3)
    return wides

  wides = pair_loop
  wait_copy(buf0, sem0)
  wait_copy(buf1, sem1)

  def lane_sum(a):
    for sh in (1, 2, 4, 8):
      perm = iota ^ sh
      a = a + jnp.take_along_axis(a, perm, axis=0, mode="promise_in_bounds")
    return a

  df = [lane_sum(w).astype(jnp.float32) for w in wides]

  zf = jnp.zeros((16,), jnp.float32)
  for j in range(OUTPAD // 16):
    vreg = zf
    for k, p in enumerate(BIN_POS):
      if j * 16 <= p < (j + 1) * 16:
        vreg = jnp.where(iota == (p - j * 16), df[k], vreg)
    row_v[pl.ds(j * 16, 16)] = vreg

  pltpu.sync_copy(row_v, out_hbm.at[wid])


def _tc_body(x_ref, out_ref, *wides):
  i = pl.program_id(0)

  @pl.when(i == 0)
  def _():
    for w in wides:
      w[...] = jnp.zeros((B, 128), jnp.int32)

  ones = jnp.ones((B, 128), jnp.int32)

  def pack_slice(j):
    sl = x_ref[:, j * 128:(j + 1) * 128].astype(jnp.int32)
    c = jnp.minimum(jnp.maximum(sl, -4), 4)
    return ones << (c * 3 + 12)

  NSL = CB // 128
  for g in range((NSL + GRP - 1) // GRP):
    lo = g * GRP
    hi = min(lo + GRP, NSL)
    acc = pack_slice(lo)
    for j in range(lo + 1, hi):
      acc = acc + pack_slice(j)
    for k, w in enumerate(wides):
      w[...] += (acc >> (3 * k)) & 7

  @pl.when(i == G_TC - 1)
  def _():
    s = [jnp.sum(w[...], axis=1, keepdims=True).astype(jnp.float32)
         for w in wides]
    iot = lax.broadcasted_iota(jnp.int32, (B, 128), 1)
    o = jnp.zeros((B, 128), jnp.float32)
    for k, p in enumerate(BIN_POS):
      o = jnp.where(iot == p, s[k], o)
    out_ref[...] = o


def _tc_call(x):
  return pl.pallas_call(
      _tc_body,
      grid=(G_TC,),
      in_specs=[pl.BlockSpec((B, CB), lambda i: (0, TC0 + i))],
      out_specs=pl.BlockSpec((B, 128), lambda i: (0, 0)),
      out_shape=jax.ShapeDtypeStruct((B, 128), jnp.float32),
      scratch_shapes=[pltpu.VMEM((B, 128), jnp.int32)] * 9,
      compiler_params=pltpu.CompilerParams(
          dimension_semantics=("arbitrary",)),
  )(x)


@jax.jit
def kernel(x):
  mesh = plsc.VectorSubcoreMesh(core_axis_name="c", subcore_axis_name="s")
  sc_out = pl.kernel(
      _hist_body,
      out_type=jax.ShapeDtypeStruct((B, OUTPAD), jnp.float32),
      mesh=mesh,
      scratch_types=[
          pltpu.VMEM((CHUNK,), jnp.float32),
          pltpu.VMEM((CHUNK,), jnp.float32),
          pltpu.VMEM((OUTPAD,), jnp.float32),
          pltpu.SemaphoreType.DMA,
          pltpu.SemaphoreType.DMA,
      ],
  )(x)
  tc_out = _tc_call(x)
  return (sc_out + tc_out)[:, :NBINS]
